# baseline (speedup 1.0000x reference)
import functools

import jax
import jax.numpy as jnp
from jax import lax
from jax.experimental import pallas as pl
from jax.experimental.pallas import tpu as pltpu
from jax.experimental.pallas import tpu_sc as plsc

N = 10000
E = 160000
D = 256
NC, NS, L = 2, 16, 16
NW = NC * NS
NP = 10112
EP = 163840
CH = 128
DH = D // 2
T1 = EP // NW // CH
T3 = EP // NS // CH
NPT = NP // NS

_mesh = plsc.VectorSubcoreMesh(core_axis_name="c", subcore_axis_name="s")


@functools.partial(
    pl.kernel,
    mesh=_mesh,
    out_type=jax.ShapeDtypeStruct((NC, NP, L), jnp.float32),
    scratch_types=[
        pltpu.VMEM_SHARED((NP, L), jnp.float32),
        pltpu.VMEM((CH, L), jnp.float32),
        pltpu.VMEM((CH,), jnp.int32),
    ],
)
def _deg_kernel(row2_hbm, z1_hbm, deg_hbm, acc_sh, ones, ix0):
    c = lax.axis_index("c")
    s = lax.axis_index("s")
    w = s * NC + c

    pltpu.sync_copy(z1_hbm.at[pl.ds(s * NPT, NPT)],
                    acc_sh.at[pl.ds(s * NPT, NPT)])
    one = jnp.full((L,), 1.0, jnp.float32)
    for i in range(CH):
        ones[i, :] = one
    plsc.subcore_barrier()

    cb = w * T1

    def group(t, carry):
        pltpu.sync_copy(row2_hbm.at[cb + t], ix0)
        pltpu.sync_copy(ones, acc_sh.at[ix0], add=True)
        return carry

    lax.fori_loop(0, T1, group, 0)
    plsc.subcore_barrier()

    pltpu.sync_copy(acc_sh.at[pl.ds(s * NPT, NPT)],
                    deg_hbm.at[c, pl.ds(s * NPT, NPT)])


def _dis_block(deg_ref):
    dg = deg_ref[...]
    deg = (jnp.sum(dg[0], axis=1) + jnp.sum(dg[1], axis=1)) * (1.0 / L)
    return jnp.where(deg == 0.0, 0.0, lax.rsqrt(deg))


def _scale_body(x_ref, deg_ref, out_ref):
    dis = _dis_block(deg_ref)
    xs = x_ref[...] * dis[:, None]
    out_ref[...] = jnp.stack([xs[:, :DH], xs[:, DH:]], axis=0)


def _scale_call(x, deg3):
    blk = 400
    return pl.pallas_call(
        _scale_body,
        grid=(N // blk,),
        in_specs=[
            pl.BlockSpec((blk, D), lambda i: (i, 0)),
            pl.BlockSpec((NC, blk, L), lambda i: (0, i, 0)),
        ],
        out_specs=pl.BlockSpec((NC, blk, DH), lambda i: (0, i, 0)),
        out_shape=jax.ShapeDtypeStruct((NC, N, DH), jnp.float32),
    )(x, deg3)


@functools.partial(
    pl.kernel,
    mesh=_mesh,
    out_type=jax.ShapeDtypeStruct((NC, NP, DH), jnp.float32),
    scratch_types=[
        pltpu.VMEM_SHARED((NP, DH), jnp.float32),
        pltpu.VMEM((CH, DH), jnp.float32),
        pltpu.VMEM((CH, DH), jnp.float32),
        pltpu.VMEM((CH, DH), jnp.float32),
        pltpu.VMEM((CH,), jnp.int32),
        pltpu.VMEM((CH,), jnp.int32),
        pltpu.VMEM((CH,), jnp.int32),
        pltpu.VMEM((CH,), jnp.int32),
        pltpu.VMEM((CH,), jnp.int32),
        pltpu.VMEM((CH,), jnp.int32),
        pltpu.SemaphoreType.DMA,
        pltpu.SemaphoreType.DMA,
        pltpu.SemaphoreType.DMA,
        pltpu.SemaphoreType.DMA,
        pltpu.SemaphoreType.DMA,
        pltpu.SemaphoreType.DMA,
    ],
)
def _agg_kernel(xs_hbm, colg_hbm, row2_hbm, z3_hbm, acc_hbm,
                acc_sh, gbuf0, gbuf1, gbuf2, gix0, gix1, gix2,
                rix0, rix1, rix2, sg0, sg1, sg2, si0, si1, si2):
    c = lax.axis_index("c")
    s = lax.axis_index("s")
    gbufs = (gbuf0, gbuf1, gbuf2)
    gixs, rixs = (gix0, gix1, gix2), (rix0, rix1, rix2)
    sgs, sis = (sg0, sg1, sg2), (si0, si1, si2)
    cb = s * T3

    pltpu.sync_copy(z3_hbm.at[pl.ds(s * NPT, NPT)],
                    acc_sh.at[pl.ds(s * NPT, NPT)])
    plsc.subcore_barrier()

    def load_idx(t, m, sync):
        if sync:
            pltpu.sync_copy(colg_hbm.at[c, cb + t], gixs[m])
            pltpu.sync_copy(row2_hbm.at[cb + t], rixs[m])
        else:
            pltpu.async_copy(colg_hbm.at[c, cb + t], gixs[m], sis[m])
            pltpu.async_copy(row2_hbm.at[cb + t], rixs[m], sis[m])

    def wait_idx(t, m):
        pltpu.make_async_copy(colg_hbm.at[c, cb + t], gixs[m],
                              sis[m]).wait()
        pltpu.make_async_copy(row2_hbm.at[cb + t], rixs[m], sis[m]).wait()

    def start_gather(m):
        pltpu.async_copy(xs_hbm.at[gixs[m]], gbufs[m], sgs[m])

    def wait_gather(m):
        pltpu.make_async_copy(xs_hbm.at[gixs[m]], gbufs[m], sgs[m]).wait()

    def scatter(m):
        pltpu.sync_copy(gbufs[m], acc_sh.at[rixs[m]], add=True)

    load_idx(0, 0, True)
    load_idx(1, 1, True)
    start_gather(0)
    start_gather(1)

    def chunk(t3, carry):
        for u in range(3):
            t = t3 * 3 + u
            m, m2 = u, (u + 2) % 3
            wait_gather(m)
            load_idx(t + 2, m2, True)
            start_gather(m2)
            scatter(m)
        return carry

    lax.fori_loop(0, T3 // 3, chunk, 0)
    wait_gather(0)
    scatter(0)
    wait_gather(1)
    scatter(1)
    plsc.subcore_barrier()

    pltpu.sync_copy(acc_sh.at[pl.ds(s * NPT, NPT)],
                    acc_hbm.at[c, pl.ds(s * NPT, NPT)])


def _out_body(acc_ref, deg_ref, w_ref, b_ref, out_ref):
    dis = _dis_block(deg_ref)
    a2 = acc_ref[...]
    h = jnp.concatenate([a2[0], a2[1]], axis=1) * dis[:, None]
    y = lax.dot_general(h, w_ref[...], (((1,), (1,)), ((), ())),
                        preferred_element_type=jnp.float32)
    out_ref[...] = y + b_ref[...]


def _out_call(acc3, deg3, W, b2):
    blk = 400
    return pl.pallas_call(
        _out_body,
        grid=(N // blk,),
        in_specs=[
            pl.BlockSpec((NC, blk, DH), lambda i: (0, i, 0)),
            pl.BlockSpec((NC, blk, L), lambda i: (0, i, 0)),
            pl.BlockSpec((D, D), lambda i: (0, 0)),
            pl.BlockSpec((1, D), lambda i: (0, 0)),
        ],
        out_specs=pl.BlockSpec((blk, D), lambda i: (i, 0)),
        out_shape=jax.ShapeDtypeStruct((N, D), jnp.float32),
    )(acc3, deg3, W, b2)


def kernel(x, edge_index, W, b):
    ei = edge_index.astype(jnp.int32)
    npad = EP + 4 * CH - E
    pad_r = (jnp.arange(npad, dtype=jnp.int32) % (NP - N)) + N
    row = jnp.concatenate([ei[0], pad_r])
    col = jnp.concatenate([ei[1], jnp.zeros((npad,), jnp.int32)])
    row2 = row.reshape(-1, CH)
    colg = jnp.stack([col, col + N]).reshape(NC, -1, CH)
    z1 = jnp.zeros((NP, L), jnp.float32)
    z3 = jnp.zeros((NP, DH), jnp.float32)

    deg3 = _deg_kernel(row2, z1)
    xs3 = _scale_call(x, deg3)
    xs_flat = xs3.reshape(NC * N, DH)
    acc3 = _agg_kernel(xs_flat, colg, row2, z3)
    return _out_call(acc3, deg3, W, b.reshape(1, D))


# speedup vs baseline: 1.0034x; 1.0034x over previous
import functools

import jax
import jax.numpy as jnp
from jax import lax
from jax.experimental import pallas as pl
from jax.experimental.pallas import tpu as pltpu
from jax.experimental.pallas import tpu_sc as plsc

N = 10000
E = 160000
D = 256
NC, NS, L = 2, 16, 16
NW = NC * NS
NP = 10240
EP = 163840
CH = 128
DH = D // 2
T1 = EP // NW // CH
T3 = EP // NS // CH

_mesh = plsc.VectorSubcoreMesh(core_axis_name="c", subcore_axis_name="s")


def _zero_vmem(buf, rows, width):
    zz = jnp.zeros((L,), jnp.float32)

    def body(i, carry):
        for k in range(width // L):
            buf[i, pl.ds(k * L, L)] = zz
        return carry

    lax.fori_loop(0, rows, body, 0)


@functools.partial(
    pl.kernel,
    mesh=_mesh,
    out_type=jax.ShapeDtypeStruct((NC, NP, L), jnp.float32),
    scratch_types=[
        pltpu.VMEM_SHARED((NP, L), jnp.float32),
        pltpu.VMEM((64, L), jnp.float32),
        pltpu.VMEM((CH, L), jnp.float32),
        pltpu.VMEM((CH,), jnp.int32),
    ],
)
def _deg_kernel(row2_hbm, deg_hbm, acc_sh, zbuf, ones, ix0):
    c = lax.axis_index("c")
    s = lax.axis_index("s")
    w = s * NC + c

    _zero_vmem(zbuf, 64, L)
    for j in range(NP // NS // 64):
        pltpu.sync_copy(zbuf, acc_sh.at[pl.ds((s * (NP // NS)) + j * 64, 64)])
    one = jnp.full((L,), 1.0, jnp.float32)
    for i in range(CH):
        ones[i, :] = one
    plsc.subcore_barrier()

    cb = w * T1

    def group(t, carry):
        pltpu.sync_copy(row2_hbm.at[cb + t], ix0)
        pltpu.sync_copy(ones, acc_sh.at[ix0], add=True)
        return carry

    lax.fori_loop(0, T1, group, 0)
    plsc.subcore_barrier()

    rb = s * (NP // NS)
    pltpu.sync_copy(acc_sh.at[pl.ds(rb, NP // NS)],
                    deg_hbm.at[c, pl.ds(rb, NP // NS)])


def _dis_block(deg_ref):
    dg = deg_ref[...]
    deg = (jnp.sum(dg[0], axis=1) + jnp.sum(dg[1], axis=1)) * (1.0 / L)
    return jnp.where(deg == 0.0, 0.0, lax.rsqrt(deg))


def _scale_body(x_ref, deg_ref, out_ref):
    dis = _dis_block(deg_ref)
    xs = x_ref[...] * dis[:, None]
    out_ref[...] = jnp.stack([xs[:, :DH], xs[:, DH:]], axis=0)


def _scale_call(x, deg3):
    blk = 400
    return pl.pallas_call(
        _scale_body,
        grid=(N // blk,),
        in_specs=[
            pl.BlockSpec((blk, D), lambda i: (i, 0)),
            pl.BlockSpec((NC, blk, L), lambda i: (0, i, 0)),
        ],
        out_specs=pl.BlockSpec((NC, blk, DH), lambda i: (0, i, 0)),
        out_shape=jax.ShapeDtypeStruct((NC, N, DH), jnp.float32),
    )(x, deg3)


@functools.partial(
    pl.kernel,
    mesh=_mesh,
    out_type=jax.ShapeDtypeStruct((NC, NP, DH), jnp.float32),
    scratch_types=[
        pltpu.VMEM_SHARED((NP, DH), jnp.float32),
        pltpu.VMEM((16, DH), jnp.float32),
        pltpu.VMEM((CH, DH), jnp.float32),
        pltpu.VMEM((CH, DH), jnp.float32),
        pltpu.VMEM((CH,), jnp.int32),
        pltpu.VMEM((CH,), jnp.int32),
        pltpu.VMEM((CH,), jnp.int32),
        pltpu.VMEM((CH,), jnp.int32),
        pltpu.SemaphoreType.DMA,
        pltpu.SemaphoreType.DMA,
        pltpu.SemaphoreType.DMA,
        pltpu.SemaphoreType.DMA,
    ],
)
def _agg_kernel(xs_hbm, colg_hbm, row2_hbm, acc_hbm,
                acc_sh, zbuf, gbuf0, gbuf1, gix0, gix1, rix0, rix1,
                sg0, sg1, si0, si1):
    c = lax.axis_index("c")
    s = lax.axis_index("s")
    gbufs = (gbuf0, gbuf1)
    gixs, rixs = (gix0, gix1), (rix0, rix1)
    sgs, sis = (sg0, sg1), (si0, si1)
    cb = s * T3

    _zero_vmem(zbuf, 16, DH)
    for j in range(NP // NS // 16):
        pltpu.sync_copy(zbuf, acc_sh.at[pl.ds(s * (NP // NS) + j * 16, 16)])
    plsc.subcore_barrier()

    pltpu.sync_copy(colg_hbm.at[c, cb], gix0)
    pltpu.sync_copy(row2_hbm.at[cb], rix0)
    pltpu.sync_copy(colg_hbm.at[c, cb + 1], gix1)
    pltpu.sync_copy(row2_hbm.at[cb + 1], rix1)
    pltpu.async_copy(xs_hbm.at[gix0], gbuf0, sg0).wait()

    def chunk(t2, carry):
        for bp in range(2):
            t = t2 * 2 + bp
            o = 1 - bp
            dg = pltpu.async_copy(xs_hbm.at[gixs[o]], gbufs[o], sgs[o])
            pltpu.sync_copy(gbufs[bp], acc_sh.at[rixs[bp]], add=True)
            d1 = pltpu.async_copy(colg_hbm.at[c, cb + t + 2], gixs[bp],
                                  sis[bp])
            d2 = pltpu.async_copy(row2_hbm.at[cb + t + 2], rixs[bp],
                                  sis[bp])
            dg.wait()
            d1.wait()
            d2.wait()
        return carry

    lax.fori_loop(0, T3 // 2, chunk, 0)
    plsc.subcore_barrier()

    rb = s * (NP // NS)
    pltpu.sync_copy(acc_sh.at[pl.ds(rb, NP // NS)],
                    acc_hbm.at[c, pl.ds(rb, NP // NS)])


def _out_body(acc_ref, deg_ref, w_ref, b_ref, out_ref):
    dis = _dis_block(deg_ref)
    a2 = acc_ref[...]
    h = jnp.concatenate([a2[0], a2[1]], axis=1) * dis[:, None]
    y = lax.dot_general(h, w_ref[...], (((1,), (1,)), ((), ())),
                        preferred_element_type=jnp.float32)
    out_ref[...] = y + b_ref[...]


def _out_call(acc3, deg3, W, b2):
    blk = 400
    return pl.pallas_call(
        _out_body,
        grid=(N // blk,),
        in_specs=[
            pl.BlockSpec((NC, blk, DH), lambda i: (0, i, 0)),
            pl.BlockSpec((NC, blk, L), lambda i: (0, i, 0)),
            pl.BlockSpec((D, D), lambda i: (0, 0)),
            pl.BlockSpec((1, D), lambda i: (0, 0)),
        ],
        out_specs=pl.BlockSpec((blk, D), lambda i: (i, 0)),
        out_shape=jax.ShapeDtypeStruct((N, D), jnp.float32),
    )(acc3, deg3, W, b2)


def kernel(x, edge_index, W, b):
    ei = edge_index.astype(jnp.int32)
    npad = EP + 2 * CH - E
    pad_r = (jnp.arange(npad, dtype=jnp.int32) % (NP - N)) + N
    row = jnp.concatenate([ei[0], pad_r])
    col = jnp.concatenate([ei[1], jnp.zeros((npad,), jnp.int32)])
    row2 = row.reshape(-1, CH)
    colg = jnp.stack([col, col + N]).reshape(NC, -1, CH)

    deg3 = _deg_kernel(row2)
    xs3 = _scale_call(x, deg3)
    xs_flat = xs3.reshape(NC * N, DH)
    acc3 = _agg_kernel(xs_flat, colg, row2)
    return _out_call(acc3, deg3, W, b.reshape(1, D))
